# Initial kernel scaffold; baseline (speedup 1.0000x reference)
import jax
import jax.numpy as jnp
from jax.experimental import pallas as pl


def kernel(x, edge_index, edge_attr, pe, batch, params):
    raise NotImplementedError("write your pallas kernel here")


# speedup vs baseline: 1.0025x; 1.0025x over previous
import functools

import jax
import jax.numpy as jnp
import numpy as np
from jax.experimental import pallas as pl
from jax.experimental.pallas import tpu as pltpu

C = 128
HEADS = 16
HD = C // HEADS


def _linear_kernel(x_ref, w_ref, b_ref, o_ref, *, act):
    y = jnp.dot(x_ref[...], w_ref[...], preferred_element_type=jnp.float32)
    y = y + b_ref[...]
    if act == "relu":
        y = jnp.maximum(y, 0.0)
    o_ref[...] = y


def _linear(x, w, b, act=None):
    M, K = x.shape
    N = w.shape[1]
    if M % 2000 == 0:
        bm = 2000
    elif M % 1000 == 0:
        bm = 1000
    else:
        bm = M
    return pl.pallas_call(
        functools.partial(_linear_kernel, act=act),
        grid=(M // bm,),
        in_specs=[
            pl.BlockSpec((bm, K), lambda i: (i, 0)),
            pl.BlockSpec((K, N), lambda i: (0, 0)),
            pl.BlockSpec((1, N), lambda i: (0, 0)),
        ],
        out_specs=pl.BlockSpec((bm, N), lambda i: (i, 0)),
        out_shape=jax.ShapeDtypeStruct((M, N), jnp.float32),
    )(x, w, b.reshape(1, -1))


def _ln_kernel(x_ref, g_ref, b_ref, o_ref, *, eps):
    x = x_ref[...]
    mu = jnp.mean(x, axis=1, keepdims=True)
    var = jnp.mean((x - mu) ** 2, axis=1, keepdims=True)
    o_ref[...] = (x - mu) / jnp.sqrt(var + eps) * g_ref[...] + b_ref[...]


def _ln(x, g, b, eps):
    M, D = x.shape
    if M % 2000 == 0:
        bm = 2000
    elif M % 1000 == 0:
        bm = 1000
    else:
        bm = M
    return pl.pallas_call(
        functools.partial(_ln_kernel, eps=eps),
        grid=(M // bm,),
        in_specs=[
            pl.BlockSpec((bm, D), lambda i: (i, 0)),
            pl.BlockSpec((1, D), lambda i: (0, 0)),
            pl.BlockSpec((1, D), lambda i: (0, 0)),
        ],
        out_specs=pl.BlockSpec((bm, D), lambda i: (i, 0)),
        out_shape=jax.ShapeDtypeStruct((M, D), jnp.float32),
    )(x, g.reshape(1, -1), b.reshape(1, -1))


def _attn_kernel(bq_ref, bk_ref, q_ref, k_ref, v_ref, o_ref):
    scale = 1.0 / np.sqrt(HD)
    mask = bq_ref[...] == bk_ref[...]
    bias = jnp.where(mask, 0.0, -1e9)
    q = q_ref[...]
    outs = []
    for h in range(HEADS):
        qh = q[:, h * HD:(h + 1) * HD]
        kh = k_ref[:, h * HD:(h + 1) * HD]
        s = jax.lax.dot_general(
            qh, kh, (((1,), (1,)), ((), ())),
            preferred_element_type=jnp.float32) * scale
        s = s + bias
        s = s - jnp.max(s, axis=1, keepdims=True)
        p = jnp.exp(s)
        p = p / jnp.sum(p, axis=1, keepdims=True)
        oh = jnp.dot(p, v_ref[:, h * HD:(h + 1) * HD],
                     preferred_element_type=jnp.float32)
        outs.append(oh)
    o_ref[...] = jnp.concatenate(outs, axis=1)


def _attn(q, k, v, bq, bk, bq_tile):
    Npad = q.shape[0]
    return pl.pallas_call(
        _attn_kernel,
        grid=(Npad // bq_tile,),
        in_specs=[
            pl.BlockSpec((bq_tile, 1), lambda i: (i, 0)),
            pl.BlockSpec((1, Npad), lambda i: (0, 0)),
            pl.BlockSpec((bq_tile, C), lambda i: (i, 0)),
            pl.BlockSpec((Npad, C), lambda i: (0, 0)),
            pl.BlockSpec((Npad, C), lambda i: (0, 0)),
        ],
        out_specs=pl.BlockSpec((bq_tile, C), lambda i: (i, 0)),
        out_shape=jax.ShapeDtypeStruct((Npad, C), jnp.float32),
    )(bq, bk, q, k, v)


def _gine_kernel(src_ref, dst_ref, ea_ref, w_ref, b_ref, x_ref, o_ref,
                 e_scr, *, be):
    step = pl.program_id(0)

    @pl.when(step == 0)
    def _():
        o_ref[...] = jnp.zeros_like(o_ref)

    e_scr[...] = (jnp.dot(ea_ref[...], w_ref[...],
                          preferred_element_type=jnp.float32) + b_ref[...])

    def body(i, _):
        s = src_ref[0, 0, i]
        d = dst_ref[0, 0, i]
        row = jnp.maximum(x_ref[s, :] + e_scr[i, :], 0.0)
        o_ref[d, :] = o_ref[d, :] + row
        return 0

    jax.lax.fori_loop(0, be, body, 0)


def _gine(src3, dst3, ea2, w, b, x):
    nsteps, _, be = src3.shape
    N = x.shape[0]
    return pl.pallas_call(
        functools.partial(_gine_kernel, be=be),
        grid=(nsteps,),
        in_specs=[
            pl.BlockSpec((1, 1, be), lambda i: (i, 0, 0),
                         memory_space=pltpu.SMEM),
            pl.BlockSpec((1, 1, be), lambda i: (i, 0, 0),
                         memory_space=pltpu.SMEM),
            pl.BlockSpec((be, C), lambda i: (i, 0)),
            pl.BlockSpec((C, C), lambda i: (0, 0)),
            pl.BlockSpec((1, C), lambda i: (0, 0)),
            pl.BlockSpec((N, C), lambda i: (0, 0)),
        ],
        out_specs=pl.BlockSpec((N, C), lambda i: (0, 0)),
        out_shape=jax.ShapeDtypeStruct((N, C), jnp.float32),
        scratch_shapes=[pltpu.VMEM((be, C), jnp.float32)],
    )(src3, dst3, ea2, w, b.reshape(1, -1), x)


def _bn_eval(x, g, b):
    return x / jnp.sqrt(1.0 + 1e-5) * g + b


def _gps_conv(xs, src3, dst3, ea2, bq, bk, Npad, lp):
    N = xs.shape[0]
    aggr = _gine(src3, dst3, ea2, lp["gine_lin_W"].T, lp["gine_lin_b"], xs)
    h0 = xs + aggr
    h = _linear(h0, lp["gine_W1"].T, lp["gine_b1"], act="relu")
    h = _linear(h, lp["gine_W2"].T, lp["gine_b2"])
    h = h + xs
    h1 = _bn_eval(h, lp["bn1_g"], lp["bn1_b"])
    qkv = _linear(xs, lp["attn_in_W"].T, lp["attn_in_b"])
    qkv = jnp.pad(qkv, ((0, Npad - N), (0, 0)))
    o = _attn(qkv[:, :C], qkv[:, C:2 * C], qkv[:, 2 * C:], bq, bk, 256)
    a = _linear(o[:N], lp["attn_out_W"].T, lp["attn_out_b"])
    a = a + xs
    h2 = _bn_eval(a, lp["bn2_g"], lp["bn2_b"])
    out = h1 + h2
    m1 = _linear(out, lp["mlp_W1"].T, lp["mlp_b1"], act="relu")
    m1 = _linear(m1, lp["mlp_W2"].T, lp["mlp_b2"])
    out = out + m1
    return _bn_eval(out, lp["bn3_g"], lp["bn3_b"])


def kernel(x, edge_index, edge_attr, pe, batch, params):
    p = params
    x = x.astype(jnp.float32)
    N = x.shape[0]
    E = edge_index.shape[1]
    Npad = ((N + 255) // 256) * 256

    be = 2000
    nsteps = E // be
    src3 = edge_index[0].reshape(nsteps, 1, be)
    dst3 = edge_index[1].reshape(nsteps, 1, be)

    bq = jnp.pad(batch, (0, Npad - N), constant_values=-1).reshape(Npad, 1)
    bk = bq.reshape(1, Npad)

    x_pe = _ln(pe, p["pe_norm_g"], p["pe_norm_b"], 1e-5)
    h = jnp.concatenate(
        [_linear(x, p["node_emb_W"].T, p["node_emb_b"]),
         _linear(x_pe, p["pe_lin_W"].T, p["pe_lin_b"])], axis=1)
    ea = _linear(edge_attr, p["edge_emb_W"].T, p["edge_emb_b"])

    for lp in p["layers"]:
        shortcut = h
        xn = _ln(h, lp["nn_g"], lp["nn_b"], 1e-6)
        ean = _ln(ea, lp["ne_g"], lp["ne_b"], 1e-6)
        z = _linear(xn, lp["fcn1_W"].T, lp["fcn1_b"])
        g_node, i_node, xs = z[:, :2 * C], z[:, 2 * C:3 * C], z[:, 3 * C:]
        ea2 = _linear(ean, lp["fce1_W"][3 * C:].T, lp["fce1_b"][3 * C:])
        xc = _gps_conv(xs, src3, dst3, ea2, bq, bk, Npad, lp)
        filt = jax.nn.gelu(g_node, approximate=False) * jnp.concatenate(
            [i_node, xc], axis=-1)
        h = _linear(filt, lp["fcn2_W"].T, lp["fcn2_b"]) + shortcut

    h = _linear(h, p["head1_W"].T, p["head1_b"], act="relu")
    h = _linear(h, p["head2_W"].T, p["head2_b"], act="relu")
    return _linear(h, p["head3_W"].T, p["head3_b"])


# speedup vs baseline: 1.3126x; 1.3093x over previous
import functools

import jax
import jax.numpy as jnp
import numpy as np
from jax.experimental import pallas as pl
from jax.experimental.pallas import tpu as pltpu

C = 128
HEADS = 16
HD = C // HEADS


def _linear_kernel(x_ref, w_ref, b_ref, o_ref, *, act):
    y = jnp.dot(x_ref[...], w_ref[...], preferred_element_type=jnp.float32)
    y = y + b_ref[...]
    if act == "relu":
        y = jnp.maximum(y, 0.0)
    o_ref[...] = y


def _linear(x, w, b, act=None):
    M, K = x.shape
    N = w.shape[1]
    if M % 2000 == 0:
        bm = 2000
    elif M % 1000 == 0:
        bm = 1000
    else:
        bm = M
    return pl.pallas_call(
        functools.partial(_linear_kernel, act=act),
        grid=(M // bm,),
        in_specs=[
            pl.BlockSpec((bm, K), lambda i: (i, 0)),
            pl.BlockSpec((K, N), lambda i: (0, 0)),
            pl.BlockSpec((1, N), lambda i: (0, 0)),
        ],
        out_specs=pl.BlockSpec((bm, N), lambda i: (i, 0)),
        out_shape=jax.ShapeDtypeStruct((M, N), jnp.float32),
    )(x, w, b.reshape(1, -1))


def _ln_kernel(x_ref, g_ref, b_ref, o_ref, *, eps):
    x = x_ref[...]
    mu = jnp.mean(x, axis=1, keepdims=True)
    var = jnp.mean((x - mu) ** 2, axis=1, keepdims=True)
    o_ref[...] = (x - mu) / jnp.sqrt(var + eps) * g_ref[...] + b_ref[...]


def _ln(x, g, b, eps):
    M, D = x.shape
    if M % 2000 == 0:
        bm = 2000
    elif M % 1000 == 0:
        bm = 1000
    else:
        bm = M
    return pl.pallas_call(
        functools.partial(_ln_kernel, eps=eps),
        grid=(M // bm,),
        in_specs=[
            pl.BlockSpec((bm, D), lambda i: (i, 0)),
            pl.BlockSpec((1, D), lambda i: (0, 0)),
            pl.BlockSpec((1, D), lambda i: (0, 0)),
        ],
        out_specs=pl.BlockSpec((bm, D), lambda i: (i, 0)),
        out_shape=jax.ShapeDtypeStruct((M, D), jnp.float32),
    )(x, g.reshape(1, -1), b.reshape(1, -1))


def _attn_kernel(bq_ref, bk_ref, q_ref, k_ref, v_ref, o_ref):
    scale = 1.0 / np.sqrt(HD)
    mask = bq_ref[...] == bk_ref[...]
    bias = jnp.where(mask, 0.0, -1e9)
    q = q_ref[...]
    outs = []
    for h in range(HEADS):
        qh = q[:, h * HD:(h + 1) * HD]
        kh = k_ref[:, h * HD:(h + 1) * HD]
        s = jax.lax.dot_general(
            qh, kh, (((1,), (1,)), ((), ())),
            preferred_element_type=jnp.float32) * scale
        s = s + bias
        s = s - jnp.max(s, axis=1, keepdims=True)
        p = jnp.exp(s)
        p = p / jnp.sum(p, axis=1, keepdims=True)
        oh = jnp.dot(p, v_ref[:, h * HD:(h + 1) * HD],
                     preferred_element_type=jnp.float32)
        outs.append(oh)
    o_ref[...] = jnp.concatenate(outs, axis=1)


def _attn(q, k, v, bq, bk, bq_tile):
    Npad = q.shape[0]
    return pl.pallas_call(
        _attn_kernel,
        grid=(Npad // bq_tile,),
        in_specs=[
            pl.BlockSpec((bq_tile, 1), lambda i: (i, 0)),
            pl.BlockSpec((1, Npad), lambda i: (0, 0)),
            pl.BlockSpec((bq_tile, C), lambda i: (i, 0)),
            pl.BlockSpec((Npad, C), lambda i: (0, 0)),
            pl.BlockSpec((Npad, C), lambda i: (0, 0)),
        ],
        out_specs=pl.BlockSpec((bq_tile, C), lambda i: (i, 0)),
        out_shape=jax.ShapeDtypeStruct((Npad, C), jnp.float32),
    )(bq, bk, q, k, v)


def _gine_kernel(src_ref, dst_ref, ea_ref, w_ref, b_ref, x_ref, o_ref,
                 e_scr, a0, a1, a2, a3, *, be, nsteps):
    step = pl.program_id(0)
    accs = (a0, a1, a2, a3)

    @pl.when(step == 0)
    def _():
        for a in accs:
            a[...] = jnp.zeros_like(a)

    e_scr[...] = (jnp.dot(ea_ref[...], w_ref[...],
                          preferred_element_type=jnp.float32) + b_ref[...])

    def body(i, _):
        for j, a in enumerate(accs):
            idx = i * 4 + j
            s = src_ref[0, 0, idx]
            d = dst_ref[0, 0, idx]
            row = jnp.maximum(x_ref[s, :] + e_scr[idx, :], 0.0)
            a[d, :] = a[d, :] + row
        return 0

    jax.lax.fori_loop(0, be // 4, body, 0)

    @pl.when(step == nsteps - 1)
    def _():
        o_ref[...] = a0[...] + a1[...] + a2[...] + a3[...]


def _gine(src3, dst3, ea2, w, b, x):
    nsteps, _, be = src3.shape
    N = x.shape[0]
    return pl.pallas_call(
        functools.partial(_gine_kernel, be=be, nsteps=nsteps),
        grid=(nsteps,),
        in_specs=[
            pl.BlockSpec((1, 1, be), lambda i: (i, 0, 0),
                         memory_space=pltpu.SMEM),
            pl.BlockSpec((1, 1, be), lambda i: (i, 0, 0),
                         memory_space=pltpu.SMEM),
            pl.BlockSpec((be, C), lambda i: (i, 0)),
            pl.BlockSpec((C, C), lambda i: (0, 0)),
            pl.BlockSpec((1, C), lambda i: (0, 0)),
            pl.BlockSpec((N, C), lambda i: (0, 0)),
        ],
        out_specs=pl.BlockSpec((N, C), lambda i: (0, 0)),
        out_shape=jax.ShapeDtypeStruct((N, C), jnp.float32),
        scratch_shapes=[pltpu.VMEM((be, C), jnp.float32)] +
                       [pltpu.VMEM((N, C), jnp.float32) for _ in range(4)],
    )(src3, dst3, ea2, w, b.reshape(1, -1), x)


def _bn_eval(x, g, b):
    return x / jnp.sqrt(1.0 + 1e-5) * g + b


def _gps_conv(xs, src3, dst3, ea2, bq, bk, Npad, lp):
    N = xs.shape[0]
    aggr = _gine(src3, dst3, ea2, lp["gine_lin_W"].T, lp["gine_lin_b"], xs)
    h0 = xs + aggr
    h = _linear(h0, lp["gine_W1"].T, lp["gine_b1"], act="relu")
    h = _linear(h, lp["gine_W2"].T, lp["gine_b2"])
    h = h + xs
    h1 = _bn_eval(h, lp["bn1_g"], lp["bn1_b"])
    qkv = _linear(xs, lp["attn_in_W"].T, lp["attn_in_b"])
    qkv = jnp.pad(qkv, ((0, Npad - N), (0, 0)))
    o = _attn(qkv[:, :C], qkv[:, C:2 * C], qkv[:, 2 * C:], bq, bk, 256)
    a = _linear(o[:N], lp["attn_out_W"].T, lp["attn_out_b"])
    a = a + xs
    h2 = _bn_eval(a, lp["bn2_g"], lp["bn2_b"])
    out = h1 + h2
    m1 = _linear(out, lp["mlp_W1"].T, lp["mlp_b1"], act="relu")
    m1 = _linear(m1, lp["mlp_W2"].T, lp["mlp_b2"])
    out = out + m1
    return _bn_eval(out, lp["bn3_g"], lp["bn3_b"])


def kernel(x, edge_index, edge_attr, pe, batch, params):
    p = params
    x = x.astype(jnp.float32)
    N = x.shape[0]
    E = edge_index.shape[1]
    Npad = ((N + 255) // 256) * 256

    be = 2000
    nsteps = E // be
    src3 = edge_index[0].reshape(nsteps, 1, be)
    dst3 = edge_index[1].reshape(nsteps, 1, be)

    bq = jnp.pad(batch, (0, Npad - N), constant_values=-1).reshape(Npad, 1)
    bk = bq.reshape(1, Npad)

    x_pe = _ln(pe, p["pe_norm_g"], p["pe_norm_b"], 1e-5)
    h = jnp.concatenate(
        [_linear(x, p["node_emb_W"].T, p["node_emb_b"]),
         _linear(x_pe, p["pe_lin_W"].T, p["pe_lin_b"])], axis=1)
    ea = _linear(edge_attr, p["edge_emb_W"].T, p["edge_emb_b"])

    for lp in p["layers"]:
        shortcut = h
        xn = _ln(h, lp["nn_g"], lp["nn_b"], 1e-6)
        ean = _ln(ea, lp["ne_g"], lp["ne_b"], 1e-6)
        z = _linear(xn, lp["fcn1_W"].T, lp["fcn1_b"])
        g_node, i_node, xs = z[:, :2 * C], z[:, 2 * C:3 * C], z[:, 3 * C:]
        ea2 = _linear(ean, lp["fce1_W"][3 * C:].T, lp["fce1_b"][3 * C:])
        xc = _gps_conv(xs, src3, dst3, ea2, bq, bk, Npad, lp)
        filt = jax.nn.gelu(g_node, approximate=False) * jnp.concatenate(
            [i_node, xc], axis=-1)
        h = _linear(filt, lp["fcn2_W"].T, lp["fcn2_b"]) + shortcut

    h = _linear(h, p["head1_W"].T, p["head1_b"], act="relu")
    h = _linear(h, p["head2_W"].T, p["head2_b"], act="relu")
    return _linear(h, p["head3_W"].T, p["head3_b"])


# speedup vs baseline: 2.4091x; 1.8354x over previous
import functools

import jax
import jax.numpy as jnp
import numpy as np
from jax.experimental import pallas as pl
from jax.experimental.pallas import tpu as pltpu

C = 128
HEADS = 16
HD = C // HEADS


def _linear_kernel(x_ref, w_ref, b_ref, o_ref, *, act):
    y = jnp.dot(x_ref[...], w_ref[...], preferred_element_type=jnp.float32)
    y = y + b_ref[...]
    if act == "relu":
        y = jnp.maximum(y, 0.0)
    o_ref[...] = y


def _linear(x, w, b, act=None):
    M, K = x.shape
    N = w.shape[1]
    if M % 2000 == 0:
        bm = 2000
    elif M % 1000 == 0:
        bm = 1000
    else:
        bm = M
    return pl.pallas_call(
        functools.partial(_linear_kernel, act=act),
        grid=(M // bm,),
        in_specs=[
            pl.BlockSpec((bm, K), lambda i: (i, 0)),
            pl.BlockSpec((K, N), lambda i: (0, 0)),
            pl.BlockSpec((1, N), lambda i: (0, 0)),
        ],
        out_specs=pl.BlockSpec((bm, N), lambda i: (i, 0)),
        out_shape=jax.ShapeDtypeStruct((M, N), jnp.float32),
    )(x, w, b.reshape(1, -1))


def _ln_kernel(x_ref, g_ref, b_ref, o_ref, *, eps):
    x = x_ref[...]
    mu = jnp.mean(x, axis=1, keepdims=True)
    var = jnp.mean((x - mu) ** 2, axis=1, keepdims=True)
    o_ref[...] = (x - mu) / jnp.sqrt(var + eps) * g_ref[...] + b_ref[...]


def _ln(x, g, b, eps):
    M, D = x.shape
    if M % 2000 == 0:
        bm = 2000
    elif M % 1000 == 0:
        bm = 1000
    else:
        bm = M
    return pl.pallas_call(
        functools.partial(_ln_kernel, eps=eps),
        grid=(M // bm,),
        in_specs=[
            pl.BlockSpec((bm, D), lambda i: (i, 0)),
            pl.BlockSpec((1, D), lambda i: (0, 0)),
            pl.BlockSpec((1, D), lambda i: (0, 0)),
        ],
        out_specs=pl.BlockSpec((bm, D), lambda i: (i, 0)),
        out_shape=jax.ShapeDtypeStruct((M, D), jnp.float32),
    )(x, g.reshape(1, -1), b.reshape(1, -1))


def _attn_kernel(bq_ref, bk_ref, q_ref, k_ref, v_ref, o_ref,
                 m_scr, l_scr, acc_scr, *, bq_tile, bw, ncb):
    scale = 1.0 / np.sqrt(HD)
    c = pl.program_id(1)

    @pl.when(c == 0)
    def _():
        m_scr[...] = jnp.full_like(m_scr, -1e30)
        l_scr[...] = jnp.zeros_like(l_scr)
        acc_scr[...] = jnp.zeros_like(acc_scr)

    active = jnp.logical_and(bk_ref[0, bw - 1] >= bq_ref[0, 0],
                             bk_ref[0, 0] <= bq_ref[bq_tile - 1, 0])

    @pl.when(active)
    def _():
        mask = bq_ref[...] == bk_ref[...]
        bias = jnp.where(mask, 0.0, -1e9)
        q = q_ref[...]
        for h in range(HEADS):
            qh = q[:, h * HD:(h + 1) * HD]
            kh = k_ref[:, h * HD:(h + 1) * HD]
            s = jax.lax.dot_general(
                qh, kh, (((1,), (1,)), ((), ())),
                preferred_element_type=jnp.float32) * scale
            s = s + bias
            m_old = m_scr[:, h:h + 1]
            m_new = jnp.maximum(m_old, jnp.max(s, axis=1, keepdims=True))
            alpha = jnp.exp(m_old - m_new)
            p = jnp.exp(s - m_new)
            l_scr[:, h:h + 1] = l_scr[:, h:h + 1] * alpha + jnp.sum(
                p, axis=1, keepdims=True)
            m_scr[:, h:h + 1] = m_new
            pv = jnp.dot(p, v_ref[:, h * HD:(h + 1) * HD],
                         preferred_element_type=jnp.float32)
            acc_scr[:, h * HD:(h + 1) * HD] = (
                acc_scr[:, h * HD:(h + 1) * HD] * alpha + pv)

    @pl.when(c == ncb - 1)
    def _():
        l = l_scr[...]
        inv = 1.0 / l
        scaled = []
        for h in range(HEADS):
            scaled.append(acc_scr[:, h * HD:(h + 1) * HD] * inv[:, h:h + 1])
        o_ref[...] = jnp.concatenate(scaled, axis=1)


def _attn(q, k, v, bq, bk, bq_tile, bw):
    Npad = q.shape[0]
    ncb = Npad // bw
    return pl.pallas_call(
        functools.partial(_attn_kernel, bq_tile=bq_tile, bw=bw, ncb=ncb),
        grid=(Npad // bq_tile, ncb),
        in_specs=[
            pl.BlockSpec((bq_tile, 1), lambda i, j: (i, 0)),
            pl.BlockSpec((1, bw), lambda i, j: (0, j)),
            pl.BlockSpec((bq_tile, C), lambda i, j: (i, 0)),
            pl.BlockSpec((bw, C), lambda i, j: (j, 0)),
            pl.BlockSpec((bw, C), lambda i, j: (j, 0)),
        ],
        out_specs=pl.BlockSpec((bq_tile, C), lambda i, j: (i, 0)),
        out_shape=jax.ShapeDtypeStruct((Npad, C), jnp.float32),
        scratch_shapes=[
            pltpu.VMEM((bq_tile, HEADS), jnp.float32),
            pltpu.VMEM((bq_tile, HEADS), jnp.float32),
            pltpu.VMEM((bq_tile, C), jnp.float32),
        ],
    )(bq, bk, q, k, v)


def _gine_kernel(src_ref, dst_ref, ea_ref, w_ref, b_ref, x_ref, o_ref,
                 e_scr, a0, a1, a2, a3, *, be, nsteps):
    step = pl.program_id(0)
    accs = (a0, a1, a2, a3)

    @pl.when(step == 0)
    def _():
        for a in accs:
            a[...] = jnp.zeros_like(a)

    e_scr[...] = (jnp.dot(ea_ref[...], w_ref[...],
                          preferred_element_type=jnp.float32) + b_ref[...])

    def body(i, _):
        for j, a in enumerate(accs):
            idx = i * 4 + j
            s = src_ref[0, 0, idx]
            d = dst_ref[0, 0, idx]
            row = jnp.maximum(x_ref[s, :] + e_scr[idx, :], 0.0)
            a[d, :] = a[d, :] + row
        return 0

    jax.lax.fori_loop(0, be // 4, body, 0)

    @pl.when(step == nsteps - 1)
    def _():
        o_ref[...] = a0[...] + a1[...] + a2[...] + a3[...]


def _gine(src3, dst3, ea2, w, b, x):
    nsteps, _, be = src3.shape
    N = x.shape[0]
    return pl.pallas_call(
        functools.partial(_gine_kernel, be=be, nsteps=nsteps),
        grid=(nsteps,),
        in_specs=[
            pl.BlockSpec((1, 1, be), lambda i: (i, 0, 0),
                         memory_space=pltpu.SMEM),
            pl.BlockSpec((1, 1, be), lambda i: (i, 0, 0),
                         memory_space=pltpu.SMEM),
            pl.BlockSpec((be, C), lambda i: (i, 0)),
            pl.BlockSpec((C, C), lambda i: (0, 0)),
            pl.BlockSpec((1, C), lambda i: (0, 0)),
            pl.BlockSpec((N, C), lambda i: (0, 0)),
        ],
        out_specs=pl.BlockSpec((N, C), lambda i: (0, 0)),
        out_shape=jax.ShapeDtypeStruct((N, C), jnp.float32),
        scratch_shapes=[pltpu.VMEM((be, C), jnp.float32)] +
                       [pltpu.VMEM((N, C), jnp.float32) for _ in range(4)],
    )(src3, dst3, ea2, w, b.reshape(1, -1), x)


def _bn_eval(x, g, b):
    return x / jnp.sqrt(1.0 + 1e-5) * g + b


def _gps_conv(xs, src3, dst3, ea2, bq, bk, Npad, lp):
    N = xs.shape[0]
    aggr = _gine(src3, dst3, ea2, lp["gine_lin_W"].T, lp["gine_lin_b"], xs)
    h0 = xs + aggr
    h = _linear(h0, lp["gine_W1"].T, lp["gine_b1"], act="relu")
    h = _linear(h, lp["gine_W2"].T, lp["gine_b2"])
    h = h + xs
    h1 = _bn_eval(h, lp["bn1_g"], lp["bn1_b"])
    qkv = _linear(xs, lp["attn_in_W"].T, lp["attn_in_b"])
    qkv = jnp.pad(qkv, ((0, Npad - N), (0, 0)))
    o = _attn(qkv[:, :C], qkv[:, C:2 * C], qkv[:, 2 * C:], bq, bk, 256, 1024)
    a = _linear(o[:N], lp["attn_out_W"].T, lp["attn_out_b"])
    a = a + xs
    h2 = _bn_eval(a, lp["bn2_g"], lp["bn2_b"])
    out = h1 + h2
    m1 = _linear(out, lp["mlp_W1"].T, lp["mlp_b1"], act="relu")
    m1 = _linear(m1, lp["mlp_W2"].T, lp["mlp_b2"])
    out = out + m1
    return _bn_eval(out, lp["bn3_g"], lp["bn3_b"])


def kernel(x, edge_index, edge_attr, pe, batch, params):
    p = params
    x = x.astype(jnp.float32)
    N = x.shape[0]
    E = edge_index.shape[1]
    Npad = ((N + 255) // 256) * 256

    be = 2000
    nsteps = E // be
    src3 = edge_index[0].reshape(nsteps, 1, be)
    dst3 = edge_index[1].reshape(nsteps, 1, be)

    bq = jnp.pad(batch, (0, Npad - N),
                 constant_values=1 << 20).reshape(Npad, 1)
    bk = bq.reshape(1, Npad)

    x_pe = _ln(pe, p["pe_norm_g"], p["pe_norm_b"], 1e-5)
    h = jnp.concatenate(
        [_linear(x, p["node_emb_W"].T, p["node_emb_b"]),
         _linear(x_pe, p["pe_lin_W"].T, p["pe_lin_b"])], axis=1)
    ea = _linear(edge_attr, p["edge_emb_W"].T, p["edge_emb_b"])

    for lp in p["layers"]:
        shortcut = h
        xn = _ln(h, lp["nn_g"], lp["nn_b"], 1e-6)
        ean = _ln(ea, lp["ne_g"], lp["ne_b"], 1e-6)
        z = _linear(xn, lp["fcn1_W"].T, lp["fcn1_b"])
        g_node, i_node, xs = z[:, :2 * C], z[:, 2 * C:3 * C], z[:, 3 * C:]
        ea2 = _linear(ean, lp["fce1_W"][3 * C:].T, lp["fce1_b"][3 * C:])
        xc = _gps_conv(xs, src3, dst3, ea2, bq, bk, Npad, lp)
        filt = jax.nn.gelu(g_node, approximate=False) * jnp.concatenate(
            [i_node, xc], axis=-1)
        h = _linear(filt, lp["fcn2_W"].T, lp["fcn2_b"]) + shortcut

    h = _linear(h, p["head1_W"].T, p["head1_b"], act="relu")
    h = _linear(h, p["head2_W"].T, p["head2_b"], act="relu")
    return _linear(h, p["head3_W"].T, p["head3_b"])
